# baseline (speedup 1.0000x reference)
import jax
import jax.numpy as jnp
from jax import lax
from jax.experimental import pallas as pl
from jax.experimental.pallas import tpu as pltpu, tpu_sc as plsc

N_ROWS = 16384
N_COLS = 200
NC = 2
NS = 16
NW = NC * NS
ROWS_PER_W = N_ROWS // NW
CHUNK_ROWS = 64
N_CHUNKS = ROWS_PER_W // CHUNK_ROWS

_OFFS = tuple(range(0, N_COLS - 16, 16)) + (N_COLS - 16,)


def _map16(x):
    in_table = plsc.bitcast(x, jnp.uint32) < jnp.uint32(1000)
    y = jnp.where(in_table, x + 2, jnp.full((16,), 1, jnp.int32))
    return jnp.where(x == 1, jnp.full((16,), 0, jnp.int32), y)


def _sc_body(in_hbm, out_hbm, in_v0, in_v1, out_v0, out_v1,
             sem_i0, sem_i1, sem_o0, sem_o1):
    wid = lax.axis_index("s") * NC + lax.axis_index("c")
    base = wid * ROWS_PER_W
    in_bufs = (in_v0, in_v1)
    out_bufs = (out_v0, out_v1)
    in_sems = (sem_i0, sem_i1)
    out_sems = (sem_o0, sem_o1)

    def in_dma(t):
        row0 = base + t * CHUNK_ROWS
        return pltpu.async_copy(
            in_hbm.at[pl.ds(row0, CHUNK_ROWS)], in_bufs[t % 2], in_sems[t % 2])

    def out_dma(t):
        row0 = base + t * CHUNK_ROWS
        return pltpu.async_copy(
            out_bufs[t % 2], out_hbm.at[pl.ds(row0, CHUNK_ROWS)],
            out_sems[t % 2])

    out_handles = [None, None]
    h_in = in_dma(0)
    for t in range(N_CHUNKS):
        h_next = in_dma(t + 1) if t + 1 < N_CHUNKS else None
        h_in.wait()
        if out_handles[t % 2] is not None:
            out_handles[t % 2].wait()
        src = in_bufs[t % 2]
        dst = out_bufs[t % 2]

        @plsc.parallel_loop(0, CHUNK_ROWS, 1, unroll=4)
        def row_body(r):
            for o in _OFFS:
                dst[r, pl.ds(o, 16)] = _map16(src[r, pl.ds(o, 16)])
        out_handles[t % 2] = out_dma(t)
        h_in = h_next
    for h in out_handles:
        if h is not None:
            h.wait()


def kernel(inputs):
    inputs = inputs.astype(jnp.int32)
    mesh = plsc.VectorSubcoreMesh(core_axis_name="c", subcore_axis_name="s")
    f = pl.kernel(
        _sc_body,
        mesh=mesh,
        out_type=jax.ShapeDtypeStruct((N_ROWS, N_COLS), jnp.int32),
        scratch_types=[
            pltpu.VMEM((CHUNK_ROWS, N_COLS), jnp.int32),
            pltpu.VMEM((CHUNK_ROWS, N_COLS), jnp.int32),
            pltpu.VMEM((CHUNK_ROWS, N_COLS), jnp.int32),
            pltpu.VMEM((CHUNK_ROWS, N_COLS), jnp.int32),
            pltpu.SemaphoreType.DMA,
            pltpu.SemaphoreType.DMA,
            pltpu.SemaphoreType.DMA,
            pltpu.SemaphoreType.DMA,
        ],
    )
    return f(inputs)


# speedup vs baseline: 1.0466x; 1.0466x over previous
import jax
import jax.numpy as jnp
from jax import lax
from jax.experimental import pallas as pl
from jax.experimental.pallas import tpu as pltpu, tpu_sc as plsc

N_ROWS = 16384
N_COLS = 200
NC = 2
NS = 16
NW = NC * NS
ROWS_PER_W = N_ROWS // NW
CHUNK_ROWS = 64
N_CHUNKS = ROWS_PER_W // CHUNK_ROWS
N_PAIRS = N_CHUNKS // 2

_OFFS = tuple(range(0, N_COLS - 16, 16)) + (N_COLS - 16,)


def _map16(x):
    in_table = plsc.bitcast(x, jnp.uint32) < jnp.uint32(1000)
    y = jnp.where(in_table, x + 2, jnp.full((16,), 1, jnp.int32))
    return jnp.where(x == 1, jnp.full((16,), 0, jnp.int32), y)


def _sc_body(in_hbm, out_hbm, in_v0, in_v1, out_v0, out_v1,
             sem_i0, sem_i1, sem_o0, sem_o1):
    wid = lax.axis_index("s") * NC + lax.axis_index("c")
    base = wid * ROWS_PER_W
    in_bufs = (in_v0, in_v1)
    out_bufs = (out_v0, out_v1)
    in_sems = (sem_i0, sem_i1)
    out_sems = (sem_o0, sem_o1)

    def start_in(t, slot):
        row0 = base + t * CHUNK_ROWS
        pltpu.async_copy(
            in_hbm.at[pl.ds(row0, CHUNK_ROWS)], in_bufs[slot], in_sems[slot])

    def start_out(t, slot):
        row0 = base + t * CHUNK_ROWS
        pltpu.async_copy(
            out_bufs[slot], out_hbm.at[pl.ds(row0, CHUNK_ROWS)],
            out_sems[slot])

    def wait_in(slot):
        pltpu.make_async_copy(
            in_hbm.at[pl.ds(base, CHUNK_ROWS)], in_bufs[slot],
            in_sems[slot]).wait()

    def wait_out(slot):
        pltpu.make_async_copy(
            out_bufs[slot], out_hbm.at[pl.ds(base, CHUNK_ROWS)],
            out_sems[slot]).wait()

    def compute(slot):
        src = in_bufs[slot]
        dst = out_bufs[slot]

        @plsc.parallel_loop(0, CHUNK_ROWS, 1, unroll=4)
        def row_body(r):
            for o in _OFFS:
                dst[r, pl.ds(o, 16)] = _map16(src[r, pl.ds(o, 16)])

    start_in(0, 0)

    def pair_body(g, carry):
        t0 = 2 * g
        start_in(t0 + 1, 1)
        wait_in(0)

        @pl.when(g > 0)
        def _():
            wait_out(0)

        compute(0)
        start_out(t0, 0)

        @pl.when(g < N_PAIRS - 1)
        def _():
            start_in(t0 + 2, 0)

        wait_in(1)

        @pl.when(g > 0)
        def _():
            wait_out(1)

        compute(1)
        start_out(t0 + 1, 1)
        return carry

    lax.fori_loop(0, N_PAIRS, pair_body, 0)
    wait_out(0)
    wait_out(1)


def kernel(inputs):
    inputs = inputs.astype(jnp.int32)
    mesh = plsc.VectorSubcoreMesh(core_axis_name="c", subcore_axis_name="s")
    f = pl.kernel(
        _sc_body,
        mesh=mesh,
        out_type=jax.ShapeDtypeStruct((N_ROWS, N_COLS), jnp.int32),
        scratch_types=[
            pltpu.VMEM((CHUNK_ROWS, N_COLS), jnp.int32),
            pltpu.VMEM((CHUNK_ROWS, N_COLS), jnp.int32),
            pltpu.VMEM((CHUNK_ROWS, N_COLS), jnp.int32),
            pltpu.VMEM((CHUNK_ROWS, N_COLS), jnp.int32),
            pltpu.SemaphoreType.DMA,
            pltpu.SemaphoreType.DMA,
            pltpu.SemaphoreType.DMA,
            pltpu.SemaphoreType.DMA,
        ],
    )
    return f(inputs)


# speedup vs baseline: 1.0519x; 1.0051x over previous
import jax
import jax.numpy as jnp
from jax import lax
from jax.experimental import pallas as pl
from jax.experimental.pallas import tpu as pltpu, tpu_sc as plsc

N_ROWS = 16384
N_COLS = 200
NC = 2
NS = 16
NW = NC * NS
ROWS_PER_W = N_ROWS // NW
CHUNK_ROWS = 128
N_CHUNKS = ROWS_PER_W // CHUNK_ROWS
N_PAIRS = N_CHUNKS // 2

_OFFS = tuple(range(0, N_COLS - 16, 16)) + (N_COLS - 16,)


def _map16(x):
    in_table = plsc.bitcast(x, jnp.uint32) < jnp.uint32(1000)
    y = jnp.where(in_table, x + 2, jnp.full((16,), 1, jnp.int32))
    return jnp.where(x == 1, jnp.full((16,), 0, jnp.int32), y)


def _sc_body(in_hbm, out_hbm, in_v0, in_v1, out_v0, out_v1,
             sem_i0, sem_i1, sem_o0, sem_o1):
    wid = lax.axis_index("s") * NC + lax.axis_index("c")
    base = wid * ROWS_PER_W
    in_bufs = (in_v0, in_v1)
    out_bufs = (out_v0, out_v1)
    in_sems = (sem_i0, sem_i1)
    out_sems = (sem_o0, sem_o1)

    def start_in(t, slot):
        row0 = base + t * CHUNK_ROWS
        pltpu.async_copy(
            in_hbm.at[pl.ds(row0, CHUNK_ROWS)], in_bufs[slot], in_sems[slot])

    def start_out(t, slot):
        row0 = base + t * CHUNK_ROWS
        pltpu.async_copy(
            out_bufs[slot], out_hbm.at[pl.ds(row0, CHUNK_ROWS)],
            out_sems[slot])

    def wait_in(slot):
        pltpu.make_async_copy(
            in_hbm.at[pl.ds(base, CHUNK_ROWS)], in_bufs[slot],
            in_sems[slot]).wait()

    def wait_out(slot):
        pltpu.make_async_copy(
            out_bufs[slot], out_hbm.at[pl.ds(base, CHUNK_ROWS)],
            out_sems[slot]).wait()

    def compute(slot):
        src = in_bufs[slot]
        dst = out_bufs[slot]

        @plsc.parallel_loop(0, CHUNK_ROWS, 1, unroll=4)
        def row_body(r):
            for o in _OFFS:
                dst[r, pl.ds(o, 16)] = _map16(src[r, pl.ds(o, 16)])

    start_in(0, 0)

    def pair_body(g, carry):
        t0 = 2 * g
        start_in(t0 + 1, 1)
        wait_in(0)

        @pl.when(g > 0)
        def _():
            wait_out(0)

        compute(0)
        start_out(t0, 0)

        @pl.when(g < N_PAIRS - 1)
        def _():
            start_in(t0 + 2, 0)

        wait_in(1)

        @pl.when(g > 0)
        def _():
            wait_out(1)

        compute(1)
        start_out(t0 + 1, 1)
        return carry

    lax.fori_loop(0, N_PAIRS, pair_body, 0)
    wait_out(0)
    wait_out(1)


def kernel(inputs):
    inputs = inputs.astype(jnp.int32)
    mesh = plsc.VectorSubcoreMesh(core_axis_name="c", subcore_axis_name="s")
    f = pl.kernel(
        _sc_body,
        mesh=mesh,
        out_type=jax.ShapeDtypeStruct((N_ROWS, N_COLS), jnp.int32),
        scratch_types=[
            pltpu.VMEM((CHUNK_ROWS, N_COLS), jnp.int32),
            pltpu.VMEM((CHUNK_ROWS, N_COLS), jnp.int32),
            pltpu.VMEM((CHUNK_ROWS, N_COLS), jnp.int32),
            pltpu.VMEM((CHUNK_ROWS, N_COLS), jnp.int32),
            pltpu.SemaphoreType.DMA,
            pltpu.SemaphoreType.DMA,
            pltpu.SemaphoreType.DMA,
            pltpu.SemaphoreType.DMA,
        ],
    )
    return f(inputs)


# speedup vs baseline: 1.3646x; 1.2972x over previous
import jax
import jax.numpy as jnp
from jax.experimental import pallas as pl


def _body(x_ref, o_ref):
    x = x_ref[...]
    in_table = (x >= 0) & (x < 1000)
    y = jnp.where(in_table, x + 2, jnp.ones_like(x))
    o_ref[...] = jnp.where(x == 1, jnp.zeros_like(x), y)


def kernel(inputs):
    inputs = inputs.astype(jnp.int32)
    n, m = inputs.shape
    block_rows = 1024
    grid = n // block_rows
    return pl.pallas_call(
        _body,
        grid=(grid,),
        in_specs=[pl.BlockSpec((block_rows, m), lambda i: (i, 0))],
        out_specs=pl.BlockSpec((block_rows, m), lambda i: (i, 0)),
        out_shape=jax.ShapeDtypeStruct((n, m), jnp.int32),
    )(inputs)
